# baseline (speedup 1.0000x reference)
import functools

import jax
import jax.numpy as jnp
from jax import lax
from jax.experimental import pallas as pl
from jax.experimental.pallas import tpu as pltpu
from jax.experimental.pallas import tpu_sc as plsc

N_NODES = 100000
D = 128
N_LABELS = 4
N_EDGES = 150000

NF = 4
FB = 32
E_PAD = 163840
EPL = E_PAD // 128
ROWS_PER_TILE = EPL // 16
WINROWS = 4
NWIN = ROWS_PER_TILE // WINROWS
WEDGES = WINROWS * 128
ACC_ROWS = 100352
ROWS_PER_SUB = ACC_ROWS // 16
ZCHUNK = 448
NZCOPY = ROWS_PER_SUB // ZCHUNK
LAST_VALID = N_NODES - 15 * ROWS_PER_SUB


def _sc_aggregate(xflat, gsrc, dst2):
    mesh = plsc.VectorSubcoreMesh(core_axis_name="c", subcore_axis_name="s")

    @functools.partial(
        pl.kernel,
        mesh=mesh,
        out_type=jax.ShapeDtypeStruct((N_LABELS, N_NODES, D), jnp.bfloat16),
        scratch_types=[
            pltpu.VMEM((3, WINROWS, 128), jnp.int32),
            pltpu.VMEM((3, WINROWS, 128), jnp.int32),
            pltpu.VMEM((2, WEDGES, FB), jnp.bfloat16),
            pltpu.VMEM((ZCHUNK, FB), jnp.bfloat16),
            pltpu.VMEM_SHARED((ACC_ROWS, FB), jnp.bfloat16),
            pltpu.SemaphoreType.DMA,
            pltpu.SemaphoreType.DMA,
            pltpu.SemaphoreType.DMA,
            pltpu.SemaphoreType.DMA,
            pltpu.SemaphoreType.DMA,
            pltpu.SemaphoreType.DMA,
        ],
        compiler_params=pltpu.CompilerParams(use_tc_tiling_on_sc=False),
    )
    def body(xflat_hbm, src_hbm, dst_hbm, out_hbm,
             idx_g, idx_s, rows, zbuf, acc,
             sem_i, sg0, sg1, ss0, ss1, sem_z):
        s = lax.axis_index("s")
        c = lax.axis_index("c")
        sg = [sg0, sg1]
        ss = [ss0, ss1]

        def zb(i, carry):
            zbuf[i, :] = jnp.zeros((FB,), jnp.bfloat16)
            return carry

        lax.fori_loop(0, ZCHUNK, zb, 0)

        def stage_idx(l, f, w, k, sem):
            row0 = s * ROWS_PER_TILE + w * WINROWS
            return [
                pltpu.async_copy(src_hbm.at[l, f, pl.ds(row0, WINROWS)], idx_g.at[k], sem),
                pltpu.async_copy(dst_hbm.at[l, pl.ds(row0, WINROWS)], idx_s.at[k], sem),
            ]

        def pair_body(k, carry):
            pair = k * 2 + c
            l = pair // NF
            f = pair % NF
            base = s * ROWS_PER_SUB

            stage0 = stage_idx(l, f, 0, 0, sem_i)
            zcps = [
                pltpu.async_copy(zbuf, acc.at[pl.ds(base + q * ZCHUNK, ZCHUNK)], sem_z)
                for q in range(NZCOPY)
            ]
            for cp in zcps:
                cp.wait()
            plsc.subcore_barrier()

            gathers = [None] * NWIN
            scatters = [None] * NWIN

            def issue_gathers(w):
                b2 = w % 2
                b3 = w % 3
                return [
                    pltpu.async_copy(
                        xflat_hbm.at[idx_g.at[b3, j]],
                        rows.at[b2, pl.ds(j * 128, 128)],
                        sg[b2],
                    )
                    for j in range(WINROWS)
                ]

            def issue_scatters(w):
                b2 = w % 2
                b3 = w % 3
                return [
                    pltpu.async_copy(
                        rows.at[b2, pl.ds(j * 128, 128)],
                        acc.at[idx_s.at[b3, j]],
                        ss[b2],
                        add=True,
                    )
                    for j in range(WINROWS)
                ]

            stages = {0: stage0}
            for w in range(NWIN):
                if w >= 2:
                    for cp in scatters[w - 2]:
                        cp.wait()
                for cp in stages[w]:
                    cp.wait()
                gathers[w] = issue_gathers(w)
                if w + 1 < NWIN:
                    stages[w + 1] = stage_idx(l, f, w + 1, (w + 1) % 3, sem_i)
                if w >= 1:
                    for cp in gathers[w - 1]:
                        cp.wait()
                    scatters[w - 1] = issue_scatters(w - 1)
            for cp in gathers[NWIN - 1]:
                cp.wait()
            scatters[NWIN - 1] = issue_scatters(NWIN - 1)
            for w in (NWIN - 2, NWIN - 1):
                for cp in scatters[w]:
                    cp.wait()
            plsc.subcore_barrier()

            @pl.when(s < 15)
            def _():
                pltpu.sync_copy(
                    acc.at[pl.ds(base, ROWS_PER_SUB)],
                    out_hbm.at[l, pl.ds(base, ROWS_PER_SUB), pl.ds(f * FB, FB)],
                )

            @pl.when(s == 15)
            def _():
                pltpu.sync_copy(
                    acc.at[pl.ds(15 * ROWS_PER_SUB, LAST_VALID)],
                    out_hbm.at[l, pl.ds(15 * ROWS_PER_SUB, LAST_VALID), pl.ds(f * FB, FB)],
                )

            return carry

        lax.fori_loop(0, N_LABELS * NF // 2, pair_body, 0)

    return body(xflat, gsrc, dst2)


def _tc_fused_matmul(x, agg, wroot, wlab, b2):
    blk = 512
    grid = (N_NODES + blk - 1) // blk

    def body(x_ref, a_ref, wr_ref, wl_ref, b_ref, o_ref):
        acc = jnp.dot(x_ref[...], wr_ref[...], preferred_element_type=jnp.float32)
        acc += b_ref[...]
        for l in range(N_LABELS):
            acc += jnp.dot(a_ref[l], wl_ref[l], preferred_element_type=jnp.float32)
        o_ref[...] = acc

    return pl.pallas_call(
        body,
        grid=(grid,),
        in_specs=[
            pl.BlockSpec((blk, D), lambda i: (i, 0)),
            pl.BlockSpec((N_LABELS, blk, D), lambda i: (0, i, 0)),
            pl.BlockSpec((D, D), lambda i: (0, 0)),
            pl.BlockSpec((N_LABELS, D, D), lambda i: (0, 0, 0)),
            pl.BlockSpec((1, D), lambda i: (0, 0)),
        ],
        out_specs=pl.BlockSpec((blk, D), lambda i: (i, 0)),
        out_shape=jax.ShapeDtypeStruct((N_NODES, D), jnp.float32),
    )(x, agg, wroot, wlab, b2)


@jax.jit
def kernel(x, edge_indices, root_W, root_b, Ws):
    ei = edge_indices.astype(jnp.int32)
    src = ei[:, 0, :]
    dst = ei[:, 1, :]
    npad = E_PAD - N_EDGES
    pad = jnp.arange(npad, dtype=jnp.int32)
    src_p = jnp.concatenate(
        [src, jnp.broadcast_to(pad % 997, (N_LABELS, npad))], axis=1)
    dst_p = jnp.concatenate(
        [dst, jnp.broadcast_to(N_NODES + pad % 352, (N_LABELS, npad))], axis=1)
    foffs = jnp.arange(NF, dtype=jnp.int32)[None, :, None]
    gsrc = (src_p[:, None, :] * NF + foffs).reshape(N_LABELS, NF, EPL, 128)
    dst2 = dst_p.reshape(N_LABELS, EPL, 128)
    xflat = x.astype(jnp.bfloat16).reshape(N_NODES * NF, FB)

    agg = _sc_aggregate(xflat, gsrc, dst2)

    wroot = root_W.T
    wlab = jnp.transpose(Ws, (0, 2, 1)).astype(jnp.bfloat16)
    b2 = root_b.reshape(1, D)
    return _tc_fused_matmul(x, agg, wroot, wlab, b2)


# speedup vs baseline: 1.2495x; 1.2495x over previous
import functools

import jax
import jax.numpy as jnp
from jax import lax
from jax.experimental import pallas as pl
from jax.experimental.pallas import tpu as pltpu
from jax.experimental.pallas import tpu_sc as plsc

N_NODES = 100000
D = 128
N_LABELS = 4
N_EDGES = 150000

NF = 4
FB = 32
E_PAD = 163840
EPL = E_PAD // 128
ROWS_PER_TILE = EPL // 16
WINROWS = 4
NWIN = ROWS_PER_TILE // WINROWS
WEDGES = WINROWS * 128
ACC_ROWS = 100352
ROWS_PER_SUB = ACC_ROWS // 16
ZCHUNK = 448
NZCOPY = ROWS_PER_SUB // ZCHUNK
LAST_VALID = N_NODES - 15 * ROWS_PER_SUB


def _sc_aggregate(xflat, gsrc, dst2):
    mesh = plsc.VectorSubcoreMesh(core_axis_name="c", subcore_axis_name="s")

    @functools.partial(
        pl.kernel,
        mesh=mesh,
        out_type=jax.ShapeDtypeStruct((N_NODES, D), jnp.bfloat16),
        scratch_types=[
            pltpu.VMEM((3, WINROWS, 128), jnp.int32),
            pltpu.VMEM((3, WINROWS, 128), jnp.int32),
            pltpu.VMEM((2, WEDGES, FB), jnp.bfloat16),
            pltpu.VMEM((ZCHUNK, FB), jnp.bfloat16),
            pltpu.VMEM_SHARED((ACC_ROWS, FB), jnp.bfloat16),
            pltpu.SemaphoreType.DMA,
            pltpu.SemaphoreType.DMA,
            pltpu.SemaphoreType.DMA,
            pltpu.SemaphoreType.DMA,
            pltpu.SemaphoreType.DMA,
            pltpu.SemaphoreType.DMA,
        ],
        compiler_params=pltpu.CompilerParams(use_tc_tiling_on_sc=False),
    )
    def body(xflat_hbm, src_hbm, dst_hbm, out_hbm,
             idx_g, idx_s, rows, zbuf, acc,
             sem_i, sg0, sg1, ss0, ss1, sem_z):
        s = lax.axis_index("s")
        c = lax.axis_index("c")
        sg = [sg0, sg1]
        ss = [ss0, ss1]

        def zb(i, carry):
            zbuf[i, :] = jnp.zeros((FB,), jnp.bfloat16)
            return carry

        lax.fori_loop(0, ZCHUNK, zb, 0)

        def stage_idx(f, w, k, sem):
            row0 = s * ROWS_PER_TILE + w * WINROWS
            return [
                pltpu.async_copy(src_hbm.at[f, pl.ds(row0, WINROWS)], idx_g.at[k], sem),
                pltpu.async_copy(dst_hbm.at[pl.ds(row0, WINROWS)], idx_s.at[k], sem),
            ]

        def pair_body(k, carry):
            f = k * 2 + c
            base = s * ROWS_PER_SUB

            stage0 = stage_idx(f, 0, 0, sem_i)
            zcps = [
                pltpu.async_copy(zbuf, acc.at[pl.ds(base + q * ZCHUNK, ZCHUNK)], sem_z)
                for q in range(NZCOPY)
            ]
            for cp in zcps:
                cp.wait()
            plsc.subcore_barrier()

            gathers = [None] * NWIN
            scatters = [None] * NWIN

            def issue_gathers(w):
                b2 = w % 2
                b3 = w % 3
                return [
                    pltpu.async_copy(
                        xflat_hbm.at[idx_g.at[b3, j]],
                        rows.at[b2, pl.ds(j * 128, 128)],
                        sg[b2],
                    )
                    for j in range(WINROWS)
                ]

            def issue_scatters(w):
                b2 = w % 2
                b3 = w % 3
                return [
                    pltpu.async_copy(
                        rows.at[b2, pl.ds(j * 128, 128)],
                        acc.at[idx_s.at[b3, j]],
                        ss[b2],
                        add=True,
                    )
                    for j in range(WINROWS)
                ]

            stages = {0: stage0}
            for w in range(NWIN):
                if w >= 2:
                    for cp in scatters[w - 2]:
                        cp.wait()
                for cp in stages[w]:
                    cp.wait()
                gathers[w] = issue_gathers(w)
                if w + 1 < NWIN:
                    stages[w + 1] = stage_idx(f, w + 1, (w + 1) % 3, sem_i)
                if w >= 1:
                    for cp in gathers[w - 1]:
                        cp.wait()
                    scatters[w - 1] = issue_scatters(w - 1)
            for cp in gathers[NWIN - 1]:
                cp.wait()
            scatters[NWIN - 1] = issue_scatters(NWIN - 1)
            for w in (NWIN - 2, NWIN - 1):
                for cp in scatters[w]:
                    cp.wait()
            plsc.subcore_barrier()

            @pl.when(s < 15)
            def _():
                pltpu.sync_copy(
                    acc.at[pl.ds(base, ROWS_PER_SUB)],
                    out_hbm.at[pl.ds(base, ROWS_PER_SUB), pl.ds(f * FB, FB)],
                )

            @pl.when(s == 15)
            def _():
                pltpu.sync_copy(
                    acc.at[pl.ds(15 * ROWS_PER_SUB, LAST_VALID)],
                    out_hbm.at[pl.ds(15 * ROWS_PER_SUB, LAST_VALID), pl.ds(f * FB, FB)],
                )

            return carry

        lax.fori_loop(0, NF // 2, pair_body, 0)

    return body(xflat, gsrc, dst2)


def _tc_fused_matmul(x, aggs, wroot, wlab, b2):
    blk = 512
    grid = (N_NODES + blk - 1) // blk

    def body(x_ref, a0_ref, a1_ref, a2_ref, a3_ref, wr_ref, wl_ref, b_ref, o_ref):
        xb = x_ref[...].astype(jnp.bfloat16)
        acc = jnp.dot(xb, wr_ref[...], preferred_element_type=jnp.float32)
        acc += b_ref[...]
        for l, a_ref in enumerate((a0_ref, a1_ref, a2_ref, a3_ref)):
            acc += jnp.dot(a_ref[...], wl_ref[l], preferred_element_type=jnp.float32)
        o_ref[...] = acc

    aspec = pl.BlockSpec((blk, D), lambda i: (i, 0))
    return pl.pallas_call(
        body,
        grid=(grid,),
        in_specs=[
            pl.BlockSpec((blk, D), lambda i: (i, 0)),
            aspec, aspec, aspec, aspec,
            pl.BlockSpec((D, D), lambda i: (0, 0)),
            pl.BlockSpec((N_LABELS, D, D), lambda i: (0, 0, 0)),
            pl.BlockSpec((1, D), lambda i: (0, 0)),
        ],
        out_specs=pl.BlockSpec((blk, D), lambda i: (i, 0)),
        out_shape=jax.ShapeDtypeStruct((N_NODES, D), jnp.float32),
    )(x, *aggs, wroot, wlab, b2)


@jax.jit
def kernel(x, edge_indices, root_W, root_b, Ws):
    ei = edge_indices.astype(jnp.int32)
    src = ei[:, 0, :]
    dst = ei[:, 1, :]
    npad = E_PAD - N_EDGES
    pad = jnp.arange(npad, dtype=jnp.int32)
    src_p = jnp.concatenate(
        [src, jnp.broadcast_to(pad % 997, (N_LABELS, npad))], axis=1)
    dst_p = jnp.concatenate(
        [dst, jnp.broadcast_to(N_NODES + pad % 352, (N_LABELS, npad))], axis=1)
    foffs = jnp.arange(NF, dtype=jnp.int32)[None, :, None]
    gsrc = (src_p[:, None, :] * NF + foffs).reshape(N_LABELS, NF, EPL, 128)
    dst2 = dst_p.reshape(N_LABELS, EPL, 128)
    xflat = x.astype(jnp.bfloat16).reshape(N_NODES * NF, FB)

    aggs = [_sc_aggregate(xflat, gsrc[l], dst2[l]) for l in range(N_LABELS)]

    wroot = root_W.T.astype(jnp.bfloat16)
    wlab = jnp.transpose(Ws, (0, 2, 1)).astype(jnp.bfloat16)
    b2 = root_b.reshape(1, D)
    return _tc_fused_matmul(x, aggs, wroot, wlab, b2)
